# Initial kernel scaffold; baseline (speedup 1.0000x reference)
import jax
import jax.numpy as jnp
from jax.experimental import pallas as pl


def kernel(query, reference_points, value, spatial_shapes, W_off, b_off, W_attn, b_attn, W_v, b_v, W_o, b_o):
    raise NotImplementedError("write your pallas kernel here")


# speedup vs baseline: 33.9067x; 33.9067x over previous
import dataclasses
import functools

import jax
import jax.numpy as jnp
from jax import lax
from jax.experimental import pallas as pl
from jax.experimental.pallas import tpu as pltpu
from jax.experimental.pallas import tpu_sc as plsc

BB, NQ, CC = 2, 50176, 96
HH, WW = 224, 224
NH, NP = 4, 4
HD = CC // NH
HDP = 32
HP, WP = HH + 1, WW + 1
R_TBL = BB * NH * HH * WW
PW = 4 * HDP

BLK = 512
NBLK = BB * NQ // BLK
NTILES = 32
QPT = BB * NQ // NTILES
QB = 16
NIT = QPT // QB
NS = NH * NP
GW = 128
NG = QB * NS // GW


def _prework_body(q_ref, rp_ref, v_ref, wox_ref, box_ref, woy_ref, boy_ref,
                  wa_ref, ba_ref, wv_ref, bv_ref,
                  vp_ref, idx_ref, w4_ref):
    pid = pl.program_id(0)
    b = pid // (NQ // BLK)
    q = q_ref[...]
    vp_ref[...] = jnp.dot(v_ref[...], wv_ref[...],
                          preferred_element_type=jnp.float32) + bv_ref[...]
    a = jnp.dot(q, wa_ref[...], preferred_element_type=jnp.float32) + ba_ref[...]
    e = jnp.exp(a)
    col = lax.broadcasted_iota(jnp.int32, (NS, NS), 0) // NP
    row = lax.broadcasted_iota(jnp.int32, (NS, NS), 1) // NP
    g = (col == row).astype(jnp.float32)
    s = jnp.dot(e, g, preferred_element_type=jnp.float32)
    aw = e / s
    offx = jnp.dot(q, wox_ref[...], preferred_element_type=jnp.float32) + box_ref[...]
    offy = jnp.dot(q, woy_ref[...], preferred_element_type=jnp.float32) + boy_ref[...]
    rx = rp_ref[:, 0:1]
    ry = rp_ref[:, 1:2]
    locx = jnp.clip(rx + offx * (0.1 / WW), 0.0, 1.0)
    locy = jnp.clip(ry + offy * (0.1 / HH), 0.0, 1.0)
    ix = jnp.clip(locx * WW - 0.5, 0.0, WW - 1.0)
    iy = jnp.clip(locy * HH - 0.5, 0.0, HH - 1.0)
    x0 = jnp.floor(ix)
    y0 = jnp.floor(iy)
    wx1 = ix - x0
    wy1 = iy - y0
    wx0 = 1.0 - wx1
    wy0 = 1.0 - wy1
    w4_ref[...] = jnp.concatenate(
        [aw * wy0 * wx0, aw * wy0 * wx1, aw * wy1 * wx0, aw * wy1 * wx1],
        axis=1)
    h = lax.broadcasted_iota(jnp.int32, (BLK, NS), 1) // NP
    r0 = (((b * NH + h) * HH + y0.astype(jnp.int32)) * WW + x0.astype(jnp.int32))
    idx_ref[...] = r0


def _run_prework(query2, rp2, value2, wox, box, woy, boy, wa, ba, wv, bv):
    full = lambda s: pl.BlockSpec(s, lambda i: tuple(0 for _ in s))
    rowblk = lambda n: pl.BlockSpec((BLK, n), lambda i: (i, 0))
    return pl.pallas_call(
        _prework_body,
        grid=(NBLK,),
        in_specs=[rowblk(CC), rowblk(2), rowblk(CC),
                  full((CC, NS)), full((NS,)), full((CC, NS)), full((NS,)),
                  full((CC, NS)), full((NS,)), full((CC, CC)), full((CC,))],
        out_specs=[rowblk(CC), rowblk(NS), rowblk(4 * NS)],
        out_shape=[jax.ShapeDtypeStruct((BB * NQ, CC), jnp.float32),
                   jax.ShapeDtypeStruct((BB * NQ, NS), jnp.int32),
                   jax.ShapeDtypeStruct((BB * NQ, 4 * NS), jnp.float32)],
    )(query2, rp2, value2, wox, box, woy, boy, wa, ba, wv, bv)


def _sc_sample_combine(table, idx_flat, w_flat):
    mesh = plsc.VectorSubcoreMesh(core_axis_name="c", subcore_axis_name="s")
    cp = pltpu.CompilerParams()
    if "needs_layout_passes" in pltpu.CompilerParams.__dataclass_fields__:
        cp = dataclasses.replace(cp, needs_layout_passes=False)

    @functools.partial(
        pl.kernel, mesh=mesh, compiler_params=cp,
        out_type=jax.ShapeDtypeStruct((BB * NQ * NH * HDP,), jnp.float32),
        scratch_types=[
            pltpu.VMEM((QB * NS,), jnp.int32),
            pltpu.VMEM((QB * NS, PW), jnp.float32),
            pltpu.VMEM((QB * 4 * NS,), jnp.float32),
            pltpu.VMEM((QB * NH * HDP,), jnp.float32),
            pltpu.SemaphoreType.DMA,
        ])
    def sck(tbl_hbm, idx_hbm, w_hbm, out_hbm, idx_v, patch_v, w_v,
            out_v, sem):
        wid = lax.axis_index("s") * 2 + lax.axis_index("c")
        base = wid * QPT

        @pl.loop(0, NIT)
        def _(it):
            qb = base + it * QB
            pltpu.sync_copy(idx_hbm.at[pl.ds(qb * NS, QB * NS)], idx_v)
            pltpu.sync_copy(w_hbm.at[pl.ds(qb * 4 * NS, QB * 4 * NS)], w_v)
            cps = [
                pltpu.async_copy(tbl_hbm.at[idx_v.at[pl.ds(gi * GW, GW)]],
                                 patch_v.at[pl.ds(gi * GW, GW)], sem)
                for gi in range(NG)
            ]
            for cp in cps:
                cp.wait()

            @pl.loop(0, QB)
            def _(q):
                for h in range(NH):
                    basev = jnp.full((16,), q * 4 * NS + h * 16, jnp.int32)
                    acc = [jnp.zeros((16,), jnp.float32) for _ in range(2)]
                    for p in range(NP):
                        s = q * NS + h * NP + p
                        for ci in range(4):
                            wb = plsc.load_gather(w_v, [basev + (p * 4 + ci)])
                            for ch in range(HDP // 16):
                                acc[ch] = acc[ch] + wb * patch_v[
                                    s, pl.ds(ci * HDP + ch * 16, 16)]
                    for ch in range(HDP // 16):
                        out_v[pl.ds(q * NH * HDP + h * HDP + ch * 16, 16)] = acc[ch]

            pltpu.sync_copy(out_v, out_hbm.at[pl.ds(qb * NH * HDP,
                                                    QB * NH * HDP)])

    return sck(table, idx_flat, w_flat)


def _proj_body(x_ref, w_ref, b_ref, o_ref):
    o_ref[...] = jnp.dot(x_ref[...], w_ref[...],
                         preferred_element_type=jnp.float32) + b_ref[...]


def _run_out_proj(samp, wo_pad, bo):
    full = lambda s: pl.BlockSpec(s, lambda i: tuple(0 for _ in s))
    return pl.pallas_call(
        _proj_body,
        grid=(NBLK,),
        in_specs=[pl.BlockSpec((BLK, NH * HDP), lambda i: (i, 0)),
                  full((NH * HDP, CC)), full((CC,))],
        out_specs=pl.BlockSpec((BLK, CC), lambda i: (i, 0)),
        out_shape=jax.ShapeDtypeStruct((BB * NQ, CC), jnp.float32),
    )(samp, wo_pad, bo)


def kernel(query, reference_points, value, spatial_shapes, W_off, b_off,
           W_attn, b_attn, W_v, b_v, W_o, b_o):
    del spatial_shapes
    q2 = query.reshape(BB * NQ, CC)
    rp2 = reference_points.reshape(BB * NQ, 2)
    v2 = value.reshape(BB * NQ, CC)
    wox, woy = W_off[:, 0::2], W_off[:, 1::2]
    box, boy = b_off[0::2], b_off[1::2]

    vp, idx, w4 = _run_prework(q2, rp2, v2, wox, box, woy, boy,
                               W_attn, b_attn, W_v, b_v)

    vsp = vp.reshape(BB, HH, WW, NH, HD)
    vsp = jnp.pad(vsp, ((0, 0), (0, 0), (0, 0), (0, 0), (0, HDP - HD)))
    vsp = vsp.transpose(0, 3, 1, 2, 4)
    vsp = jnp.pad(vsp, ((0, 0), (0, 0), (0, 1), (0, 1), (0, 0)), mode="edge")
    table = jnp.concatenate(
        [vsp[:, :, 0:HH, 0:WW, :], vsp[:, :, 0:HH, 1:WP, :],
         vsp[:, :, 1:HP, 0:WW, :], vsp[:, :, 1:HP, 1:WP, :]],
        axis=-1).reshape(R_TBL, PW)

    w4 = w4.reshape(-1, 4, NH, NP).transpose(0, 2, 3, 1).reshape(-1)

    samp = _sc_sample_combine(table, idx.reshape(-1), w4)
    samp = samp.reshape(BB * NQ, NH * HDP)

    wo = W_o.reshape(NH, HD, CC)
    wo_pad = jnp.pad(wo, ((0, 0), (0, HDP - HD), (0, 0))).reshape(NH * HDP, CC)
    out = _run_out_proj(samp, wo_pad, b_o)
    return out.reshape(BB, NQ, CC)


# speedup vs baseline: 70.8859x; 2.0906x over previous
import dataclasses
import functools

import jax
import jax.numpy as jnp
from jax import lax
from jax.experimental import pallas as pl
from jax.experimental.pallas import tpu as pltpu
from jax.experimental.pallas import tpu_sc as plsc

BB, NQ, CC = 2, 50176, 96
HH, WW = 224, 224
NH, NP = 4, 4
HD = CC // NH
HDP = 32
HP, WP = HH + 1, WW + 1
R_TBL = BB * NH * HH * WW
PW = 4 * HDP

BLK = 512
NBLK = BB * NQ // BLK
NTILES = 32
QPT = BB * NQ // NTILES
QB = 16
NIT = QPT // QB
NS = NH * NP
GW = 128
NG = QB * NS // GW


def _prework_body(q_ref, rp_ref, v_ref, wox_ref, box_ref, woy_ref, boy_ref,
                  wa_ref, ba_ref, wv_ref, bv_ref,
                  vp_ref, idx_ref, w4_ref):
    pid = pl.program_id(0)
    b = pid // (NQ // BLK)
    q = q_ref[...]
    vp_ref[...] = jnp.dot(v_ref[...], wv_ref[...],
                          preferred_element_type=jnp.float32) + bv_ref[...]
    a = jnp.dot(q, wa_ref[...], preferred_element_type=jnp.float32) + ba_ref[...]
    e = jnp.exp(a)
    col = lax.broadcasted_iota(jnp.int32, (NS, NS), 0) // NP
    row = lax.broadcasted_iota(jnp.int32, (NS, NS), 1) // NP
    g = (col == row).astype(jnp.float32)
    s = jnp.dot(e, g, preferred_element_type=jnp.float32)
    aw = e / s
    offx = jnp.dot(q, wox_ref[...], preferred_element_type=jnp.float32) + box_ref[...]
    offy = jnp.dot(q, woy_ref[...], preferred_element_type=jnp.float32) + boy_ref[...]
    rx = rp_ref[:, 0:1]
    ry = rp_ref[:, 1:2]
    locx = jnp.clip(rx + offx * (0.1 / WW), 0.0, 1.0)
    locy = jnp.clip(ry + offy * (0.1 / HH), 0.0, 1.0)
    ix = jnp.clip(locx * WW - 0.5, 0.0, WW - 1.0)
    iy = jnp.clip(locy * HH - 0.5, 0.0, HH - 1.0)
    x0 = jnp.floor(ix)
    y0 = jnp.floor(iy)
    wx1 = ix - x0
    wy1 = iy - y0
    wx0 = 1.0 - wx1
    wy0 = 1.0 - wy1
    wcat = jnp.concatenate(
        [aw * wy0 * wx0, aw * wy0 * wx1, aw * wy1 * wx0, aw * wy1 * wx1],
        axis=1)
    jo = lax.broadcasted_iota(jnp.int32, (4 * NS, 4 * NS), 0)
    jn = lax.broadcasted_iota(jnp.int32, (4 * NS, 4 * NS), 1)
    perm = ((jo % NS // NP) * 16 + (jo % NP) * 4 + jo // NS == jn)
    w4_ref[...] = jnp.dot(wcat, perm.astype(jnp.float32),
                          preferred_element_type=jnp.float32)
    h = lax.broadcasted_iota(jnp.int32, (BLK, NS), 1) // NP
    r0 = (((b * NH + h) * HH + y0.astype(jnp.int32)) * WW + x0.astype(jnp.int32))
    idx_ref[...] = r0


def _run_prework(query2, rp2, value2, wox, box, woy, boy, wa, ba, wv, bv):
    full = lambda s: pl.BlockSpec(s, lambda i: tuple(0 for _ in s))
    rowblk = lambda n: pl.BlockSpec((BLK, n), lambda i: (i, 0))
    return pl.pallas_call(
        _prework_body,
        grid=(NBLK,),
        in_specs=[rowblk(CC), rowblk(2), rowblk(CC),
                  full((CC, NS)), full((NS,)), full((CC, NS)), full((NS,)),
                  full((CC, NS)), full((NS,)), full((CC, CC)), full((CC,))],
        out_specs=[rowblk(CC), rowblk(NS), rowblk(4 * NS)],
        out_shape=[jax.ShapeDtypeStruct((BB * NQ, CC), jnp.float32),
                   jax.ShapeDtypeStruct((BB * NQ, NS), jnp.int32),
                   jax.ShapeDtypeStruct((BB * NQ, 4 * NS), jnp.float32)],
    )(query2, rp2, value2, wox, box, woy, boy, wa, ba, wv, bv)


YB = 8
NYB = HH // YB


def _table_body(vp_hbm, tbl_hbm, in_v, out_v, sem_in, sem_out):
    b = pl.program_id(0)
    yb = pl.program_id(1)
    row0 = b * NQ + yb * YB * WW
    pltpu.make_async_copy(vp_hbm.at[pl.ds(row0, YB * WW)],
                          in_v.at[pl.ds(0, YB * WW)], sem_in).start()
    start9 = jnp.where(yb == NYB - 1, row0 + (YB - 1) * WW, row0 + YB * WW)
    pltpu.make_async_copy(vp_hbm.at[pl.ds(start9, WW)],
                          in_v.at[pl.ds(YB * WW, WW)], sem_in).start()
    pltpu.make_async_copy(vp_hbm.at[pl.ds(row0, YB * WW)],
                          in_v.at[pl.ds(0, YB * WW)], sem_in).wait()
    pltpu.make_async_copy(vp_hbm.at[pl.ds(start9, WW)],
                          in_v.at[pl.ds(YB * WW, WW)], sem_in).wait()
    x = in_v[...].reshape(YB + 1, WW, CC)
    for h in range(NH):
        c = x[:, :, h * HD:(h + 1) * HD]
        c = jnp.concatenate(
            [c, jnp.zeros((YB + 1, WW, HDP - HD), jnp.float32)], axis=2)
        r0 = c[0:YB]
        r1 = c[1:YB + 1]
        r0s = jnp.concatenate([r0[:, 1:WW, :], r0[:, WW - 1:WW, :]], axis=1)
        r1s = jnp.concatenate([r1[:, 1:WW, :], r1[:, WW - 1:WW, :]], axis=1)
        out_v[...] = jnp.concatenate([r0, r0s, r1, r1s], axis=2)
        cp = pltpu.make_async_copy(
            out_v, tbl_hbm.at[pl.ds((b * NH + h) * HH + yb * YB, YB)],
            sem_out)
        cp.start()
        cp.wait()


def _run_table_build(vp):
    return pl.pallas_call(
        _table_body,
        grid=(BB, NYB),
        in_specs=[pl.BlockSpec(memory_space=pl.ANY)],
        out_specs=pl.BlockSpec(memory_space=pl.ANY),
        out_shape=jax.ShapeDtypeStruct((BB * NH * HH, WW, PW), jnp.float32),
        scratch_shapes=[pltpu.VMEM(((YB + 1) * WW, CC), jnp.float32),
                        pltpu.VMEM((YB, WW, PW), jnp.float32),
                        pltpu.SemaphoreType.DMA,
                        pltpu.SemaphoreType.DMA],
    )(vp)


def _sc_sample_combine(table, idx_flat, w_flat):
    mesh = plsc.VectorSubcoreMesh(core_axis_name="c", subcore_axis_name="s")
    cp = pltpu.CompilerParams()
    if "needs_layout_passes" in pltpu.CompilerParams.__dataclass_fields__:
        cp = dataclasses.replace(cp, needs_layout_passes=False)

    @functools.partial(
        pl.kernel, mesh=mesh, compiler_params=cp,
        out_type=jax.ShapeDtypeStruct((BB * NQ * NH * HDP,), jnp.float32),
        scratch_types=[
            pltpu.VMEM((QB * NS,), jnp.int32),
            pltpu.VMEM((QB * NS, PW), jnp.float32),
            pltpu.VMEM((QB * 4 * NS,), jnp.float32),
            pltpu.VMEM((QB * NH * HDP,), jnp.float32),
            pltpu.SemaphoreType.DMA,
        ])
    def sck(tbl_hbm, idx_hbm, w_hbm, out_hbm, idx_v, patch_v, w_v,
            out_v, sem):
        wid = lax.axis_index("s") * 2 + lax.axis_index("c")
        base = wid * QPT

        @pl.loop(0, NIT)
        def _(it):
            qb = base + it * QB
            pltpu.sync_copy(idx_hbm.at[pl.ds(qb * NS, QB * NS)], idx_v)
            pltpu.sync_copy(w_hbm.at[pl.ds(qb * 4 * NS, QB * 4 * NS)], w_v)
            cps = [
                pltpu.async_copy(tbl_hbm.at[idx_v.at[pl.ds(gi * GW, GW)]],
                                 patch_v.at[pl.ds(gi * GW, GW)], sem)
                for gi in range(NG)
            ]
            for cp in cps:
                cp.wait()

            @pl.loop(0, QB)
            def _(q):
                for h in range(NH):
                    basev = jnp.full((16,), q * 4 * NS + h * 16, jnp.int32)
                    acc = [jnp.zeros((16,), jnp.float32) for _ in range(2)]
                    for p in range(NP):
                        s = q * NS + h * NP + p
                        for ci in range(4):
                            wb = plsc.load_gather(w_v, [basev + (p * 4 + ci)])
                            for ch in range(HDP // 16):
                                acc[ch] = acc[ch] + wb * patch_v[
                                    s, pl.ds(ci * HDP + ch * 16, 16)]
                    for ch in range(HDP // 16):
                        out_v[pl.ds(q * NH * HDP + h * HDP + ch * 16, 16)] = acc[ch]

            pltpu.sync_copy(out_v, out_hbm.at[pl.ds(qb * NH * HDP,
                                                    QB * NH * HDP)])

    return sck(table, idx_flat, w_flat)


def _proj_body(x_ref, w_ref, b_ref, o_ref):
    o_ref[...] = jnp.dot(x_ref[...], w_ref[...],
                         preferred_element_type=jnp.float32) + b_ref[...]


def _run_out_proj(samp, wo_pad, bo):
    full = lambda s: pl.BlockSpec(s, lambda i: tuple(0 for _ in s))
    return pl.pallas_call(
        _proj_body,
        grid=(NBLK,),
        in_specs=[pl.BlockSpec((BLK, NH * HDP), lambda i: (i, 0)),
                  full((NH * HDP, CC)), full((CC,))],
        out_specs=pl.BlockSpec((BLK, CC), lambda i: (i, 0)),
        out_shape=jax.ShapeDtypeStruct((BB * NQ, CC), jnp.float32),
    )(samp, wo_pad, bo)


def kernel(query, reference_points, value, spatial_shapes, W_off, b_off,
           W_attn, b_attn, W_v, b_v, W_o, b_o):
    del spatial_shapes
    q2 = query.reshape(BB * NQ, CC)
    rp2 = reference_points.reshape(BB * NQ, 2)
    v2 = value.reshape(BB * NQ, CC)
    wox, woy = W_off[:, 0::2], W_off[:, 1::2]
    box, boy = b_off[0::2], b_off[1::2]

    vp, idx, w4 = _run_prework(q2, rp2, v2, wox, box, woy, boy,
                               W_attn, b_attn, W_v, b_v)

    table = _run_table_build(vp).reshape(R_TBL, PW)

    samp = _sc_sample_combine(table, idx.reshape(-1), w4.reshape(-1))
    samp = samp.reshape(BB * NQ, NH * HDP)

    wo = W_o.reshape(NH, HD, CC)
    wo_pad = jnp.pad(wo, ((0, 0), (0, HDP - HD), (0, 0))).reshape(NH * HDP, CC)
    out = _run_out_proj(samp, wo_pad, b_o)
    return out.reshape(BB, NQ, CC)


# speedup vs baseline: 127.4562x; 1.7980x over previous
import dataclasses
import functools

import jax
import jax.numpy as jnp
from jax import lax
from jax.experimental import pallas as pl
from jax.experimental.pallas import tpu as pltpu
from jax.experimental.pallas import tpu_sc as plsc

BB, NQ, CC = 2, 50176, 96
HH, WW = 224, 224
NH, NP = 4, 4
HD = CC // NH
HDP = 32
R_TBL = BB * NH * HH * WW
PW = 4 * HDP

BLK = 2048
NBLK = BB * NQ // BLK
NTILES = 32
QPT = BB * NQ // NTILES
QB = 16
NIT = QPT // QB
NS = NH * NP
GW = 128
NG = QB * NS // GW


def _prework_body(q_ref, rp_ref, v_ref, wox_ref, box_ref, woy_ref, boy_ref,
                  wa_ref, ba_ref, wv_ref, bv_ref,
                  vp_ref, idx_ref, w4_ref):
    pid = pl.program_id(0)
    q = q_ref[...]
    vp_ref[...] = jnp.dot(v_ref[...], wv_ref[...],
                          preferred_element_type=jnp.float32) + bv_ref[...]
    a = jnp.dot(q, wa_ref[...], preferred_element_type=jnp.float32) + ba_ref[...]
    e = jnp.exp(a)
    col = lax.broadcasted_iota(jnp.int32, (NS, NS), 0) // NP
    row = lax.broadcasted_iota(jnp.int32, (NS, NS), 1) // NP
    g = (col == row).astype(jnp.float32)
    s = jnp.dot(e, g, preferred_element_type=jnp.float32)
    aw = e / s
    offx = jnp.dot(q, wox_ref[...], preferred_element_type=jnp.float32) + box_ref[...]
    offy = jnp.dot(q, woy_ref[...], preferred_element_type=jnp.float32) + boy_ref[...]
    rx = rp_ref[:, 0:1]
    ry = rp_ref[:, 1:2]
    locx = jnp.clip(rx + offx * (0.1 / WW), 0.0, 1.0)
    locy = jnp.clip(ry + offy * (0.1 / HH), 0.0, 1.0)
    ix = jnp.clip(locx * WW - 0.5, 0.0, WW - 1.0)
    iy = jnp.clip(locy * HH - 0.5, 0.0, HH - 1.0)
    x0 = jnp.floor(ix)
    y0 = jnp.floor(iy)
    wx1 = ix - x0
    wy1 = iy - y0
    wx0 = 1.0 - wx1
    wy0 = 1.0 - wy1
    wcat = jnp.concatenate(
        [aw * wy0 * wx0, aw * wy0 * wx1, aw * wy1 * wx0, aw * wy1 * wx1],
        axis=1)
    jo = lax.broadcasted_iota(jnp.int32, (4 * NS, 4 * NS), 0)
    jn = lax.broadcasted_iota(jnp.int32, (4 * NS, 4 * NS), 1)
    perm = ((jo % NS // NP) * 16 + (jo % NP) * 4 + jo // NS == jn)
    w4_ref[...] = jnp.dot(wcat, perm.astype(jnp.float32),
                          preferred_element_type=jnp.float32)
    h = lax.broadcasted_iota(jnp.int32, (BLK, NS), 1) // NP
    bq = (pid * BLK + lax.broadcasted_iota(jnp.int32, (BLK, NS), 0)) // NQ
    r0 = (((bq * NH + h) * HH + y0.astype(jnp.int32)) * WW
          + x0.astype(jnp.int32))
    idx_ref[...] = r0


def _run_prework(query2, rp2, value2, wox, box, woy, boy, wa, ba, wv, bv):
    full = lambda s: pl.BlockSpec(s, lambda i: tuple(0 for _ in s))
    rowblk = lambda n: pl.BlockSpec((BLK, n), lambda i: (i, 0))
    return pl.pallas_call(
        _prework_body,
        grid=(NBLK,),
        in_specs=[rowblk(CC), rowblk(2), rowblk(CC),
                  full((CC, NS)), full((NS,)), full((CC, NS)), full((NS,)),
                  full((CC, NS)), full((NS,)), full((CC, CC)), full((CC,))],
        out_specs=[rowblk(CC), rowblk(NS), rowblk(4 * NS)],
        out_shape=[jax.ShapeDtypeStruct((BB * NQ, CC), jnp.float32),
                   jax.ShapeDtypeStruct((BB * NQ, NS), jnp.int32),
                   jax.ShapeDtypeStruct((BB * NQ, 4 * NS), jnp.float32)],
    )(query2, rp2, value2, wox, box, woy, boy, wa, ba, wv, bv)


YB = 8
NYB = HH // YB
NSTEP = BB * NYB


def _table_body(vp_hbm, tbl_hbm, in_a, in_b, out_a, out_b, sem_in, sem_out):
    g = pl.program_id(0)
    in_bufs = (in_a, in_b)
    out_bufs = (out_a, out_b)

    def in_copies(s, buf):
        b = s // NYB
        yb = s % NYB
        row0 = b * NQ + yb * YB * WW
        start9 = jnp.where(yb == NYB - 1, row0 + (YB - 1) * WW,
                           row0 + YB * WW)
        return (pltpu.make_async_copy(vp_hbm.at[pl.ds(row0, YB * WW)],
                                      buf.at[pl.ds(0, YB * WW)], sem_in),
                pltpu.make_async_copy(vp_hbm.at[pl.ds(start9, WW)],
                                      buf.at[pl.ds(YB * WW, WW)], sem_in))

    def out_row(s, h):
        return ((s // NYB) * NH + h) * HH + (s % NYB) * YB

    @pl.when(g == 0)
    def _():
        for cp in in_copies(0, in_a):
            cp.start()

    def do_step(s, par):
        for cp in in_copies(s, in_bufs[par]):
            cp.wait()

        @pl.when(s + 1 < NSTEP)
        def _():
            for cp in in_copies(s + 1, in_bufs[1 - par]):
                cp.start()

        x = in_bufs[par][...].reshape(YB + 1, WW, CC)
        for h in range(NH):
            ob = out_bufs[h % 2]
            if h >= 2:
                pltpu.make_async_copy(
                    ob, tbl_hbm.at[pl.ds(out_row(s, h - 2), YB)],
                    sem_out).wait()
            else:
                @pl.when(s > 0)
                def _():
                    pltpu.make_async_copy(
                        ob, tbl_hbm.at[pl.ds(out_row(s - 1, h + 2), YB)],
                        sem_out).wait()
            c = x[:, :, h * HD:(h + 1) * HD]
            c = jnp.concatenate(
                [c, jnp.zeros((YB + 1, WW, HDP - HD), jnp.float32)], axis=2)
            r0 = c[0:YB]
            r1 = c[1:YB + 1]
            r0s = jnp.concatenate([r0[:, 1:WW, :], r0[:, WW - 1:WW, :]],
                                  axis=1)
            r1s = jnp.concatenate([r1[:, 1:WW, :], r1[:, WW - 1:WW, :]],
                                  axis=1)
            ob[...] = jnp.concatenate([r0, r0s, r1, r1s], axis=2)
            pltpu.make_async_copy(
                ob, tbl_hbm.at[pl.ds(out_row(s, h), YB)], sem_out).start()

    do_step(2 * g, 0)
    do_step(2 * g + 1, 1)

    @pl.when(g == NSTEP // 2 - 1)
    def _():
        for h in (2, 3):
            pltpu.make_async_copy(
                out_bufs[h % 2],
                tbl_hbm.at[pl.ds(out_row(NSTEP - 1, h), YB)],
                sem_out).wait()


def _run_table_build(vp):
    return pl.pallas_call(
        _table_body,
        grid=(NSTEP // 2,),
        in_specs=[pl.BlockSpec(memory_space=pl.ANY)],
        out_specs=pl.BlockSpec(memory_space=pl.ANY),
        out_shape=jax.ShapeDtypeStruct((BB * NH * HH, WW, PW), jnp.float32),
        scratch_shapes=[pltpu.VMEM(((YB + 1) * WW, CC), jnp.float32),
                        pltpu.VMEM(((YB + 1) * WW, CC), jnp.float32),
                        pltpu.VMEM((YB, WW, PW), jnp.float32),
                        pltpu.VMEM((YB, WW, PW), jnp.float32),
                        pltpu.SemaphoreType.DMA,
                        pltpu.SemaphoreType.DMA],
    )(vp)


def _sc_sample_combine(table, idx_flat, w_flat):
    mesh = plsc.VectorSubcoreMesh(core_axis_name="c", subcore_axis_name="s")
    cp = pltpu.CompilerParams()
    if "needs_layout_passes" in pltpu.CompilerParams.__dataclass_fields__:
        cp = dataclasses.replace(cp, needs_layout_passes=False)

    @functools.partial(
        pl.kernel, mesh=mesh, compiler_params=cp,
        out_type=jax.ShapeDtypeStruct((BB * NQ * NH * HDP,), jnp.float32),
        scratch_types=[
            pltpu.VMEM((QB * NS,), jnp.int32),
            pltpu.VMEM((QB * NS,), jnp.int32),
            pltpu.VMEM((QB * NS, PW), jnp.float32),
            pltpu.VMEM((QB * NS, PW), jnp.float32),
            pltpu.VMEM((QB * 4 * NS,), jnp.float32),
            pltpu.VMEM((QB * 4 * NS,), jnp.float32),
            pltpu.VMEM((QB * NH * HDP,), jnp.float32),
            pltpu.VMEM((QB * NH * HDP,), jnp.float32),
            pltpu.SemaphoreType.DMA,
            pltpu.SemaphoreType.DMA,
            pltpu.SemaphoreType.DMA,
            pltpu.SemaphoreType.DMA,
            pltpu.SemaphoreType.DMA,
            pltpu.SemaphoreType.DMA,
        ])
    def sck(tbl_hbm, idx_hbm, w_hbm, out_hbm, idx_v0, idx_v1, patch_v0,
            patch_v1, w_v0, w_v1, out_v0, out_v1,
            gsem0, gsem1, wsem0, wsem1, osem0, osem1):
        wid = lax.axis_index("s") * 2 + lax.axis_index("c")
        base = wid * QPT
        idx_v = (idx_v0, idx_v1)
        patch_v = (patch_v0, patch_v1)
        w_v = (w_v0, w_v1)
        out_v = (out_v0, out_v1)
        gsem = (gsem0, gsem1)
        wsem = (wsem0, wsem1)
        osem = (osem0, osem1)

        def stage1(i, p):
            qb = base + i * QB
            pltpu.sync_copy(idx_hbm.at[pl.ds(qb * NS, QB * NS)], idx_v[p])
            pltpu.async_copy(w_hbm.at[pl.ds(qb * 4 * NS, QB * 4 * NS)],
                             w_v[p], wsem[p])
            for gi in range(NG):
                pltpu.async_copy(
                    tbl_hbm.at[idx_v[p].at[pl.ds(gi * GW, GW)]],
                    patch_v[p].at[pl.ds(gi * GW, GW)], gsem[p])

        def stage2(i, p, wait_out):
            qb = base + i * QB
            for gi in range(NG):
                pltpu.make_async_copy(
                    tbl_hbm.at[idx_v[p].at[pl.ds(gi * GW, GW)]],
                    patch_v[p].at[pl.ds(gi * GW, GW)], gsem[p]).wait()
            pltpu.make_async_copy(
                w_hbm.at[pl.ds(qb * 4 * NS, QB * 4 * NS)], w_v[p],
                wsem[p]).wait()
            qb2 = base + (i - 2) * QB
            @pl.when(wait_out)
            def _():
                pltpu.make_async_copy(
                    out_v[p], out_hbm.at[pl.ds(qb2 * NH * HDP,
                                               QB * NH * HDP)],
                    osem[p]).wait()

            @pl.loop(0, QB)
            def _(q):
                for h in range(NH):
                    basev = jnp.full((16,), q * 4 * NS + h * 16, jnp.int32)
                    acc = [jnp.zeros((16,), jnp.float32) for _ in range(2)]
                    for pp in range(NP):
                        s = q * NS + h * NP + pp
                        for ci in range(4):
                            wb = plsc.load_gather(
                                w_v[p], [basev + (pp * 4 + ci)])
                            for ch in range(HDP // 16):
                                acc[ch] = acc[ch] + wb * patch_v[p][
                                    s, pl.ds(ci * HDP + ch * 16, 16)]
                    for ch in range(HDP // 16):
                        out_v[p][pl.ds(q * NH * HDP + h * HDP + ch * 16,
                                       16)] = acc[ch]

            pltpu.async_copy(out_v[p],
                             out_hbm.at[pl.ds(qb * NH * HDP, QB * NH * HDP)],
                             osem[p])

        stage1(0, 0)

        @pl.loop(0, NIT, step=2)
        def _(k):
            stage1(k + 1, 1)
            stage2(k, 0, wait_out=k >= 2)

            @pl.when(k + 2 < NIT)
            def _():
                stage1(k + 2, 0)

            stage2(k + 1, 1, wait_out=k >= 2)

        for p in range(2):
            qbf = base + (NIT - 2 + p) * QB
            pltpu.make_async_copy(
                out_v[p], out_hbm.at[pl.ds(qbf * NH * HDP,
                                           QB * NH * HDP)],
                osem[p]).wait()

    return sck(table, idx_flat, w_flat)


def _proj_body(x_ref, w_ref, b_ref, o_ref):
    o_ref[...] = jnp.dot(x_ref[...], w_ref[...],
                         preferred_element_type=jnp.float32) + b_ref[...]


def _run_out_proj(samp, wo_pad, bo):
    full = lambda s: pl.BlockSpec(s, lambda i: tuple(0 for _ in s))
    return pl.pallas_call(
        _proj_body,
        grid=(NBLK,),
        in_specs=[pl.BlockSpec((BLK, NH * HDP), lambda i: (i, 0)),
                  full((NH * HDP, CC)), full((CC,))],
        out_specs=pl.BlockSpec((BLK, CC), lambda i: (i, 0)),
        out_shape=jax.ShapeDtypeStruct((BB * NQ, CC), jnp.float32),
    )(samp, wo_pad, bo)


def kernel(query, reference_points, value, spatial_shapes, W_off, b_off,
           W_attn, b_attn, W_v, b_v, W_o, b_o):
    del spatial_shapes
    q2 = query.reshape(BB * NQ, CC)
    rp2 = reference_points.reshape(BB * NQ, 2)
    v2 = value.reshape(BB * NQ, CC)
    wox, woy = W_off[:, 0::2], W_off[:, 1::2]
    box, boy = b_off[0::2], b_off[1::2]

    vp, idx, w4 = _run_prework(q2, rp2, v2, wox, box, woy, boy,
                               W_attn, b_attn, W_v, b_v)

    table = _run_table_build(vp).reshape(R_TBL, PW)

    samp = _sc_sample_combine(table, idx.reshape(-1), w4.reshape(-1))
    samp = samp.reshape(BB * NQ, NH * HDP)

    wo = W_o.reshape(NH, HD, CC)
    wo_pad = jnp.pad(wo, ((0, 0), (0, HDP - HD), (0, 0))).reshape(NH * HDP, CC)
    out = _run_out_proj(samp, wo_pad, b_o)
    return out.reshape(BB, NQ, CC)


# speedup vs baseline: 132.0276x; 1.0359x over previous
import dataclasses
import functools

import jax
import jax.numpy as jnp
from jax import lax
from jax.experimental import pallas as pl
from jax.experimental.pallas import tpu as pltpu
from jax.experimental.pallas import tpu_sc as plsc

BB, NQ, CC = 2, 50176, 96
HH, WW = 224, 224
NH, NP = 4, 4
HD = CC // NH
HDP = 32
R_TBL = BB * NH * HH * WW
PW = 4 * HDP

BLK = 2048
NBLK = BB * NQ // BLK
NTILES = 32
QPT = BB * NQ // NTILES
QB = 16
NIT = QPT // QB
NS = NH * NP
GW = 128
NG = QB * NS // GW


def _prework_body(q_ref, rp_ref, v_ref, wox_ref, box_ref, woy_ref, boy_ref,
                  wa_ref, ba_ref, wv_ref, bv_ref,
                  vp_ref, aux_ref):
    pid = pl.program_id(0)
    q = q_ref[...]
    vp_ref[...] = jnp.dot(v_ref[...], wv_ref[...],
                          preferred_element_type=jnp.float32) + bv_ref[...]
    a = jnp.dot(q, wa_ref[...], preferred_element_type=jnp.float32) + ba_ref[...]
    e = jnp.exp(a)
    col = lax.broadcasted_iota(jnp.int32, (NS, NS), 0) // NP
    row = lax.broadcasted_iota(jnp.int32, (NS, NS), 1) // NP
    g = (col == row).astype(jnp.float32)
    s = jnp.dot(e, g, preferred_element_type=jnp.float32)
    aw = e / s
    offx = jnp.dot(q, wox_ref[...], preferred_element_type=jnp.float32) + box_ref[...]
    offy = jnp.dot(q, woy_ref[...], preferred_element_type=jnp.float32) + boy_ref[...]
    rx = rp_ref[:, 0:1]
    ry = rp_ref[:, 1:2]
    locx = jnp.clip(rx + offx * (0.1 / WW), 0.0, 1.0)
    locy = jnp.clip(ry + offy * (0.1 / HH), 0.0, 1.0)
    ix = jnp.clip(locx * WW - 0.5, 0.0, WW - 1.0)
    iy = jnp.clip(locy * HH - 0.5, 0.0, HH - 1.0)
    x0 = jnp.floor(ix)
    y0 = jnp.floor(iy)
    wx1 = ix - x0
    wy1 = iy - y0
    wx0 = 1.0 - wx1
    wy0 = 1.0 - wy1
    wcat = jnp.concatenate(
        [aw * wy0 * wx0, aw * wy0 * wx1, aw * wy1 * wx0, aw * wy1 * wx1],
        axis=1)
    jo = lax.broadcasted_iota(jnp.int32, (4 * NS, 4 * NS), 0)
    jn = lax.broadcasted_iota(jnp.int32, (4 * NS, 4 * NS), 1)
    perm = ((jo % NS // NP) * 16 + (jo % NP) * 4 + jo // NS == jn)
    w4 = jnp.dot(wcat, perm.astype(jnp.float32),
                 preferred_element_type=jnp.float32)
    h = lax.broadcasted_iota(jnp.int32, (BLK, NS), 1) // NP
    bq = (pid * BLK + lax.broadcasted_iota(jnp.int32, (BLK, NS), 0)) // NQ
    r0 = (((bq * NH + h) * HH + y0.astype(jnp.int32)) * WW
          + x0.astype(jnp.int32))
    aux_ref[...] = jnp.concatenate(
        [w4, lax.bitcast_convert_type(r0, jnp.float32),
         jnp.zeros((BLK, PW - 5 * NS), jnp.float32)], axis=1)


def _run_prework(query2, rp2, value2, wox, box, woy, boy, wa, ba, wv, bv):
    full = lambda s: pl.BlockSpec(s, lambda i: tuple(0 for _ in s))
    rowblk = lambda n: pl.BlockSpec((BLK, n), lambda i: (i, 0))
    return pl.pallas_call(
        _prework_body,
        grid=(NBLK,),
        in_specs=[rowblk(CC), rowblk(2), rowblk(CC),
                  full((CC, NS)), full((NS,)), full((CC, NS)), full((NS,)),
                  full((CC, NS)), full((NS,)), full((CC, CC)), full((CC,))],
        out_specs=[rowblk(CC), rowblk(PW)],
        out_shape=[jax.ShapeDtypeStruct((BB * NQ, CC), jnp.float32),
                   jax.ShapeDtypeStruct((BB * NQ, PW), jnp.float32)],
    )(query2, rp2, value2, wox, box, woy, boy, wa, ba, wv, bv)


YB = 16
NYB = HH // YB
NSTEP = BB * NYB


def _table_body(vp_hbm, tbl_hbm, in_a, in_b, out_a, out_b, sem_in, sem_out):
    g = pl.program_id(0)
    in_bufs = (in_a, in_b)
    out_bufs = (out_a, out_b)

    def in_copies(s, buf):
        b = s // NYB
        yb = s % NYB
        row0 = b * NQ + yb * YB * WW
        start9 = jnp.where(yb == NYB - 1, row0 + (YB - 1) * WW,
                           row0 + YB * WW)
        return (pltpu.make_async_copy(vp_hbm.at[pl.ds(row0, YB * WW)],
                                      buf.at[pl.ds(0, YB * WW)], sem_in),
                pltpu.make_async_copy(vp_hbm.at[pl.ds(start9, WW)],
                                      buf.at[pl.ds(YB * WW, WW)], sem_in))

    def out_row(s, h):
        return ((s // NYB) * NH + h) * HH + (s % NYB) * YB

    @pl.when(g == 0)
    def _():
        for cp in in_copies(0, in_a):
            cp.start()

    def do_step(s, par):
        for cp in in_copies(s, in_bufs[par]):
            cp.wait()

        @pl.when(s + 1 < NSTEP)
        def _():
            for cp in in_copies(s + 1, in_bufs[1 - par]):
                cp.start()

        x = in_bufs[par][...].reshape(YB + 1, WW, CC)
        for h in range(NH):
            ob = out_bufs[h % 2]
            if h >= 2:
                pltpu.make_async_copy(
                    ob, tbl_hbm.at[pl.ds(out_row(s, h - 2), YB)],
                    sem_out).wait()
            else:
                @pl.when(s > 0)
                def _():
                    pltpu.make_async_copy(
                        ob, tbl_hbm.at[pl.ds(out_row(s - 1, h + 2), YB)],
                        sem_out).wait()
            c = x[:, :, h * HD:(h + 1) * HD]
            c = jnp.concatenate(
                [c, jnp.zeros((YB + 1, WW, HDP - HD), jnp.float32)], axis=2)
            r0 = c[0:YB]
            r1 = c[1:YB + 1]
            r0s = jnp.concatenate([r0[:, 1:WW, :], r0[:, WW - 1:WW, :]],
                                  axis=1)
            r1s = jnp.concatenate([r1[:, 1:WW, :], r1[:, WW - 1:WW, :]],
                                  axis=1)
            ob[...] = jnp.concatenate([r0, r0s, r1, r1s], axis=2)
            pltpu.make_async_copy(
                ob, tbl_hbm.at[pl.ds(out_row(s, h), YB)], sem_out).start()

    do_step(2 * g, 0)
    do_step(2 * g + 1, 1)

    @pl.when(g == NSTEP // 2 - 1)
    def _():
        for h in (2, 3):
            pltpu.make_async_copy(
                out_bufs[h % 2],
                tbl_hbm.at[pl.ds(out_row(NSTEP - 1, h), YB)],
                sem_out).wait()


def _run_table_build(vp):
    return pl.pallas_call(
        _table_body,
        grid=(NSTEP // 2,),
        in_specs=[pl.BlockSpec(memory_space=pl.ANY)],
        out_specs=pl.BlockSpec(memory_space=pl.ANY),
        out_shape=jax.ShapeDtypeStruct((BB * NH * HH, WW, PW), jnp.float32),
        scratch_shapes=[pltpu.VMEM(((YB + 1) * WW, CC), jnp.float32),
                        pltpu.VMEM(((YB + 1) * WW, CC), jnp.float32),
                        pltpu.VMEM((YB, WW, PW), jnp.float32),
                        pltpu.VMEM((YB, WW, PW), jnp.float32),
                        pltpu.SemaphoreType.DMA,
                        pltpu.SemaphoreType.DMA],
    )(vp)


def _sc_sample_combine(table, aux):
    mesh = plsc.VectorSubcoreMesh(core_axis_name="c", subcore_axis_name="s")
    cp = pltpu.CompilerParams()
    if "needs_layout_passes" in pltpu.CompilerParams.__dataclass_fields__:
        cp = dataclasses.replace(cp, needs_layout_passes=False)

    @functools.partial(
        pl.kernel, mesh=mesh, compiler_params=cp,
        out_type=jax.ShapeDtypeStruct((BB * NQ, NH * HDP), jnp.float32),
        scratch_types=[
            pltpu.VMEM((QB, PW), jnp.float32),
            pltpu.VMEM((QB, PW), jnp.float32),
            pltpu.VMEM((QB * NS,), jnp.int32),
            pltpu.VMEM((QB * NS,), jnp.int32),
            pltpu.VMEM((QB * NS, PW), jnp.float32),
            pltpu.VMEM((QB * NS, PW), jnp.float32),
            pltpu.VMEM((QB, NH * HDP), jnp.float32),
            pltpu.VMEM((QB, NH * HDP), jnp.float32),
            pltpu.SemaphoreType.DMA,
            pltpu.SemaphoreType.DMA,
            pltpu.SemaphoreType.DMA,
            pltpu.SemaphoreType.DMA,
        ])
    def sck(tbl_hbm, aux_hbm, out_hbm, aux_v0, aux_v1, idx_v0, idx_v1,
            patch_v0, patch_v1, out_v0, out_v1, gsem0, gsem1, osem0, osem1):
        wid = lax.axis_index("s") * 2 + lax.axis_index("c")
        base = wid * QPT
        aux_v = (aux_v0, aux_v1)
        idx_v = (idx_v0, idx_v1)
        patch_v = (patch_v0, patch_v1)
        out_v = (out_v0, out_v1)
        gsem = (gsem0, gsem1)
        osem = (osem0, osem1)

        def stage1(i, p):
            qb = base + i * QB
            pltpu.sync_copy(aux_hbm.at[pl.ds(qb, QB)], aux_v[p])
            @pl.loop(0, QB)
            def _(q):
                iv = plsc.bitcast(aux_v[p][q, pl.ds(4 * NS, NS)], jnp.int32)
                idx_v[p][pl.ds(q * NS, NS)] = iv
            for gi in range(NG):
                pltpu.async_copy(
                    tbl_hbm.at[idx_v[p].at[pl.ds(gi * GW, GW)]],
                    patch_v[p].at[pl.ds(gi * GW, GW)], gsem[p])

        wcols = [jnp.full((16,), c, jnp.int32) for c in range(4 * NS)]

        def stage2(i, p, wait_out):
            qb = base + i * QB
            for gi in range(NG):
                pltpu.make_async_copy(
                    tbl_hbm.at[idx_v[p].at[pl.ds(gi * GW, GW)]],
                    patch_v[p].at[pl.ds(gi * GW, GW)], gsem[p]).wait()
            qb2 = base + (i - 2) * QB
            @pl.when(wait_out)
            def _():
                pltpu.make_async_copy(
                    out_v[p], out_hbm.at[pl.ds(qb2, QB)], osem[p]).wait()

            @pl.loop(0, QB)
            def _(q):
                qv = jnp.full((16,), q, jnp.int32)
                for h in range(NH):
                    acc = [jnp.zeros((16,), jnp.float32) for _ in range(2)]
                    for pp in range(NP):
                        s = q * NS + h * NP + pp
                        for ci in range(4):
                            wb = plsc.load_gather(
                                aux_v[p], [qv, wcols[h * 16 + pp * 4 + ci]])
                            for ch in range(HDP // 16):
                                acc[ch] = acc[ch] + wb * patch_v[p][
                                    s, pl.ds(ci * HDP + ch * 16, 16)]
                    for ch in range(HDP // 16):
                        out_v[p][q, pl.ds(h * HDP + ch * 16, 16)] = acc[ch]

            pltpu.async_copy(out_v[p], out_hbm.at[pl.ds(qb, QB)], osem[p])

        stage1(0, 0)

        @pl.loop(0, NIT, step=2)
        def _(k):
            stage1(k + 1, 1)
            stage2(k, 0, wait_out=k >= 2)

            @pl.when(k + 2 < NIT)
            def _():
                stage1(k + 2, 0)

            stage2(k + 1, 1, wait_out=k >= 2)

        for p in range(2):
            qbf = base + (NIT - 2 + p) * QB
            pltpu.make_async_copy(
                out_v[p], out_hbm.at[pl.ds(qbf, QB)], osem[p]).wait()

    return sck(table, aux)


def _proj_body(x_ref, w_ref, b_ref, o_ref):
    o_ref[...] = jnp.dot(x_ref[...], w_ref[...],
                         preferred_element_type=jnp.float32) + b_ref[...]


def _run_out_proj(samp, wo_pad, bo):
    full = lambda s: pl.BlockSpec(s, lambda i: tuple(0 for _ in s))
    return pl.pallas_call(
        _proj_body,
        grid=(NBLK,),
        in_specs=[pl.BlockSpec((BLK, NH * HDP), lambda i: (i, 0)),
                  full((NH * HDP, CC)), full((CC,))],
        out_specs=pl.BlockSpec((BLK, CC), lambda i: (i, 0)),
        out_shape=jax.ShapeDtypeStruct((BB * NQ, CC), jnp.float32),
    )(samp, wo_pad, bo)


def kernel(query, reference_points, value, spatial_shapes, W_off, b_off,
           W_attn, b_attn, W_v, b_v, W_o, b_o):
    del spatial_shapes
    q2 = query.reshape(BB * NQ, CC)
    rp2 = reference_points.reshape(BB * NQ, 2)
    v2 = value.reshape(BB * NQ, CC)
    wox, woy = W_off[:, 0::2], W_off[:, 1::2]
    box, boy = b_off[0::2], b_off[1::2]

    vp, aux = _run_prework(q2, rp2, v2, wox, box, woy, boy,
                           W_attn, b_attn, W_v, b_v)

    table = _run_table_build(vp).reshape(R_TBL, PW)

    samp = _sc_sample_combine(table, aux)

    wo = W_o.reshape(NH, HD, CC)
    wo_pad = jnp.pad(wo, ((0, 0), (0, HDP - HD), (0, 0))).reshape(NH * HDP, CC)
    out = _run_out_proj(samp, wo_pad, b_o)
    return out.reshape(BB, NQ, CC)
